# Initial kernel scaffold; baseline (speedup 1.0000x reference)
import jax
import jax.numpy as jnp
from jax.experimental import pallas as pl


def kernel(actions, tables, W1, b1, W2, b2):
    raise NotImplementedError("write your pallas kernel here")


# speedup vs baseline: 290.5992x; 290.5992x over previous
import math

import jax
import jax.numpy as jnp
from jax.experimental import pallas as pl
from jax.experimental.pallas import tpu as pltpu

B_ = 16384
A_ = 26
NB_ = 20
D_ = 32
OUT_ = 128
H_ = (A_ * D_) // 2
C_ = A_ * NB_
AD_ = A_ * D_

BT = 512
NT = B_ // BT

_INV_SQRT2 = 1.0 / math.sqrt(2.0)


def _gelu(x):
    return 0.5 * x * (1.0 + jax.lax.erf(x * _INV_SQRT2))


def _minmax_body(act_ref, mm_ref):
    t = pl.program_id(0)
    act = act_ref[...]
    mn = jnp.min(act, axis=0, keepdims=True)
    mx = jnp.max(act, axis=0, keepdims=True)
    cur = jnp.concatenate([mn, -mx], axis=0)

    @pl.when(t == 0)
    def _init():
        mm_ref[...] = cur

    @pl.when(t != 0)
    def _acc():
        mm_ref[...] = jnp.minimum(mm_ref[...], cur)


def _main_body(tlin_ref, mm_ref, act_ref, tab_ref, W1_ref, b1_ref, W2_ref,
               b2_ref, out_ref, E_ref, M_ref):
    t = pl.program_id(0)

    @pl.when(t == 0)
    def _prep():
        er = jax.lax.broadcasted_iota(jnp.int32, (A_, C_), 0)
        ec = jax.lax.broadcasted_iota(jnp.int32, (A_, C_), 1)
        E_ref[...] = jnp.where(ec // NB_ == er, 1.0, 0.0)
        dr = jax.lax.broadcasted_iota(jnp.int32, (D_, AD_), 0)
        dc = jax.lax.broadcasted_iota(jnp.int32, (D_, AD_), 1)
        erep = jnp.where(dc % D_ == dr, 1.0, 0.0)
        mr = jax.lax.broadcasted_iota(jnp.int32, (C_, AD_), 0)
        mc = jax.lax.broadcasted_iota(jnp.int32, (C_, AD_), 1)
        mask = jnp.where(mr // NB_ == mc // D_, 1.0, 0.0)
        t520 = jnp.dot(tab_ref[...], erep,
                       preferred_element_type=jnp.float32) * mask
        M_ref[...] = jnp.dot(t520, W1_ref[...],
                             preferred_element_type=jnp.float32)

    act = act_ref[...]
    mn = mm_ref[0:1, :]
    diff = (-mm_ref[1:2, :]) - mn
    cnt = jnp.zeros_like(act)
    for k in range(NB_ + 1):
        th = mn + diff * tlin_ref[0, k]
        cnt = cnt + jnp.where(th < act, 1.0, 0.0)
    binv = jnp.clip(cnt - 1.0, 0.0, float(NB_ - 1))
    bin_e = jnp.dot(binv, E_ref[...], preferred_element_type=jnp.float32)
    cidx = jax.lax.broadcasted_iota(jnp.int32, (1, C_), 1)
    jmod = (cidx % NB_).astype(jnp.float32)
    onehot = jnp.where(bin_e == jmod, 1.0, 0.0)
    hpre = jnp.dot(onehot, M_ref[...],
                   preferred_element_type=jnp.float32) + b1_ref[...]
    h = _gelu(hpre)
    o = jnp.dot(h, W2_ref[...], preferred_element_type=jnp.float32)
    out_ref[...] = _gelu(o + b2_ref[...])


def kernel(actions, tables, W1, b1, W2, b2):
    tab520 = tables.reshape(C_, D_)
    tlin = jnp.linspace(0.0, 1.0, NB_ + 1, dtype=jnp.float32).reshape(1, NB_ + 1)
    b1r = b1.reshape(1, H_)
    b2r = b2.reshape(1, OUT_)

    mm = pl.pallas_call(
        _minmax_body,
        grid=(NT,),
        in_specs=[pl.BlockSpec((BT, A_), lambda t: (t, 0))],
        out_specs=pl.BlockSpec((2, A_), lambda t: (0, 0)),
        out_shape=jax.ShapeDtypeStruct((2, A_), jnp.float32),
        compiler_params=pltpu.CompilerParams(
            dimension_semantics=("arbitrary",)),
    )(actions)

    out = pl.pallas_call(
        _main_body,
        grid=(NT,),
        in_specs=[
            pl.BlockSpec((1, NB_ + 1), lambda t: (0, 0)),
            pl.BlockSpec((2, A_), lambda t: (0, 0)),
            pl.BlockSpec((BT, A_), lambda t: (t, 0)),
            pl.BlockSpec((C_, D_), lambda t: (0, 0)),
            pl.BlockSpec((AD_, H_), lambda t: (0, 0)),
            pl.BlockSpec((1, H_), lambda t: (0, 0)),
            pl.BlockSpec((H_, OUT_), lambda t: (0, 0)),
            pl.BlockSpec((1, OUT_), lambda t: (0, 0)),
        ],
        out_specs=pl.BlockSpec((BT, OUT_), lambda t: (t, 0)),
        out_shape=jax.ShapeDtypeStruct((B_, OUT_), jnp.float32),
        scratch_shapes=[
            pltpu.VMEM((A_, C_), jnp.float32),
            pltpu.VMEM((C_, H_), jnp.float32),
        ],
        compiler_params=pltpu.CompilerParams(
            dimension_semantics=("arbitrary",)),
    )(tlin, mm, actions, tab520, W1, b1r, W2, b2r)
    return out


# speedup vs baseline: 362.8948x; 1.2488x over previous
import math

import jax
import jax.numpy as jnp
from jax.experimental import pallas as pl
from jax.experimental.pallas import tpu as pltpu

B_ = 16384
A_ = 26
NB_ = 20
D_ = 32
OUT_ = 128
H_ = (A_ * D_) // 2
C_ = A_ * NB_
AD_ = A_ * D_

BT = 512
NT = B_ // BT

_INV_SQRT2 = 1.0 / math.sqrt(2.0)


def _gelu(x):
    return 0.5 * x * (1.0 + jax.lax.erf(x * _INV_SQRT2))


def _minmax_body(act_ref, mm_ref):
    t = pl.program_id(0)
    act = act_ref[...]
    mn = jnp.min(act, axis=0, keepdims=True)
    mx = jnp.max(act, axis=0, keepdims=True)
    cur = jnp.concatenate([mn, -mx], axis=0)

    @pl.when(t == 0)
    def _init():
        mm_ref[...] = cur

    @pl.when(t != 0)
    def _acc():
        mm_ref[...] = jnp.minimum(mm_ref[...], cur)


def _main_body(tlin_ref, mm_ref, act_ref, tab_ref, W1_ref, b1_ref, W2_ref,
               b2_ref, out_ref, E_ref, M_ref, LU_ref):
    t = pl.program_id(0)

    @pl.when(t == 0)
    def _prep():
        er = jax.lax.broadcasted_iota(jnp.int32, (A_, C_), 0)
        ec = jax.lax.broadcasted_iota(jnp.int32, (A_, C_), 1)
        E_ref[...] = jnp.where(ec // NB_ == er, 1.0, 0.0)
        dr = jax.lax.broadcasted_iota(jnp.int32, (D_, AD_), 0)
        dc = jax.lax.broadcasted_iota(jnp.int32, (D_, AD_), 1)
        erep = jnp.where(dc % D_ == dr, 1.0, 0.0)
        mr = jax.lax.broadcasted_iota(jnp.int32, (C_, AD_), 0)
        mc = jax.lax.broadcasted_iota(jnp.int32, (C_, AD_), 1)
        mask = jnp.where(mr // NB_ == mc // D_, 1.0, 0.0)
        t520 = jnp.dot(tab_ref[...], erep,
                       preferred_element_type=jnp.float32) * mask
        M_ref[...] = jnp.dot(t520, W1_ref[...],
                             preferred_element_type=jnp.float32)
        mn = mm_ref[0:1, :]
        diff = (-mm_ref[1:2, :]) - mn
        kr = jax.lax.broadcasted_iota(jnp.int32, (NB_ + 1, C_), 0)
        kc = jax.lax.broadcasted_iota(jnp.int32, (NB_ + 1, C_), 1)
        jm = kc % NB_
        Plo = jnp.where(jm == kr, 1.0, 0.0)
        Pup = jnp.where(jm + 1 == kr, 1.0, 0.0)
        tl = jnp.dot(tlin_ref[...], Plo, preferred_element_type=jnp.float32)
        tu = jnp.dot(tlin_ref[...], Pup, preferred_element_type=jnp.float32)
        mn_e = jnp.dot(mn, E_ref[...], preferred_element_type=jnp.float32)
        df_e = jnp.dot(diff, E_ref[...], preferred_element_type=jnp.float32)
        jrow = jax.lax.broadcasted_iota(jnp.int32, (1, C_), 1) % NB_
        LU_ref[0:1, :] = jnp.where(jrow == 0, -jnp.inf, mn_e + df_e * tl)
        LU_ref[1:2, :] = jnp.where(jrow == NB_ - 1, jnp.inf, mn_e + df_e * tu)

    act = act_ref[...]
    act_e = jnp.dot(act, E_ref[...], preferred_element_type=jnp.float32)
    onehot = jnp.where(
        (act_e > LU_ref[0:1, :]) & (act_e <= LU_ref[1:2, :]), 1.0, 0.0)
    hpre = jnp.dot(onehot, M_ref[...],
                   preferred_element_type=jnp.float32) + b1_ref[...]
    h = _gelu(hpre)
    o = jnp.dot(h, W2_ref[...], preferred_element_type=jnp.float32)
    out_ref[...] = _gelu(o + b2_ref[...])


def kernel(actions, tables, W1, b1, W2, b2):
    tab520 = tables.reshape(C_, D_)
    tlin = jnp.linspace(0.0, 1.0, NB_ + 1, dtype=jnp.float32).reshape(1, NB_ + 1)
    b1r = b1.reshape(1, H_)
    b2r = b2.reshape(1, OUT_)

    mm = pl.pallas_call(
        _minmax_body,
        grid=(NT,),
        in_specs=[pl.BlockSpec((BT, A_), lambda t: (t, 0))],
        out_specs=pl.BlockSpec((2, A_), lambda t: (0, 0)),
        out_shape=jax.ShapeDtypeStruct((2, A_), jnp.float32),
        compiler_params=pltpu.CompilerParams(
            dimension_semantics=("arbitrary",)),
    )(actions)

    out = pl.pallas_call(
        _main_body,
        grid=(NT,),
        in_specs=[
            pl.BlockSpec((1, NB_ + 1), lambda t: (0, 0)),
            pl.BlockSpec((2, A_), lambda t: (0, 0)),
            pl.BlockSpec((BT, A_), lambda t: (t, 0)),
            pl.BlockSpec((C_, D_), lambda t: (0, 0)),
            pl.BlockSpec((AD_, H_), lambda t: (0, 0)),
            pl.BlockSpec((1, H_), lambda t: (0, 0)),
            pl.BlockSpec((H_, OUT_), lambda t: (0, 0)),
            pl.BlockSpec((1, OUT_), lambda t: (0, 0)),
        ],
        out_specs=pl.BlockSpec((BT, OUT_), lambda t: (t, 0)),
        out_shape=jax.ShapeDtypeStruct((B_, OUT_), jnp.float32),
        scratch_shapes=[
            pltpu.VMEM((A_, C_), jnp.float32),
            pltpu.VMEM((C_, H_), jnp.float32),
            pltpu.VMEM((2, C_), jnp.float32),
        ],
        compiler_params=pltpu.CompilerParams(
            dimension_semantics=("arbitrary",)),
    )(tlin, mm, actions, tab520, W1, b1r, W2, b2r)
    return out
